# Initial kernel scaffold; baseline (speedup 1.0000x reference)
import jax
import jax.numpy as jnp
from jax.experimental import pallas as pl


def kernel(words, vectors, W, b):
    raise NotImplementedError("write your pallas kernel here")


# speedup vs baseline: 2.7018x; 2.7018x over previous
import functools

import jax
import jax.numpy as jnp
from jax import lax
from jax.experimental import pallas as pl
from jax.experimental.pallas import tpu as pltpu
from jax.experimental.pallas import tpu_sc as plsc

VOCAB = 1000000
D = 32
C = 100
B = 16384
H = 50

NW = 32
BPW = B // NW
CB = 32
NCHUNK = BPW // CB
IDX_PER_CHUNK = CB * H
GB = 80
NGATHER = IDX_PER_CHUNK // GB


def _sc_gather_mean(words_flat, vectors):
    info = plsc.get_sparse_core_info()
    nc = info.num_cores
    mesh = plsc.VectorSubcoreMesh(core_axis_name="c", subcore_axis_name="s")

    @functools.partial(
        pl.kernel,
        mesh=mesh,
        out_type=jax.ShapeDtypeStruct((B, D), jnp.float32),
        compiler_params=pltpu.CompilerParams(use_tc_tiling_on_sc=False),
        scratch_types=[
            pltpu.VMEM((IDX_PER_CHUNK,), jnp.int32),
            pltpu.VMEM((IDX_PER_CHUNK, D), jnp.float32),
            pltpu.VMEM((CB, D), jnp.float32),
            pltpu.SemaphoreType.DMA,
        ],
    )
    def k(words_hbm, table_hbm, feat_hbm, idx_v, rows_v, feat_v, sem):
        wid = lax.axis_index("s") * nc + lax.axis_index("c")
        base_elem = wid * BPW

        def chunk_body(ch, carry):
            elem0 = base_elem + ch * CB
            idx_base = pl.multiple_of(elem0 * H, 8)
            pltpu.sync_copy(words_hbm.at[pl.ds(idx_base, IDX_PER_CHUNK)], idx_v)
            copies = [
                pltpu.async_copy(
                    table_hbm.at[idx_v.at[pl.ds(j * GB, GB)]],
                    rows_v.at[pl.ds(j * GB, GB)],
                    sem,
                )
                for j in range(NGATHER)
            ]
            for cp in copies:
                cp.wait()

            def elem_body(e, carry2):
                row0 = e * H
                a0 = jnp.zeros((16,), jnp.float32)
                a1 = jnp.zeros((16,), jnp.float32)
                for j in range(H):
                    a0 = a0 + rows_v[row0 + j, pl.ds(0, 16)]
                    a1 = a1 + rows_v[row0 + j, pl.ds(16, 16)]
                feat_v[e, pl.ds(0, 16)] = a0 * (1.0 / H)
                feat_v[e, pl.ds(16, 16)] = a1 * (1.0 / H)
                return carry2

            lax.fori_loop(0, CB, elem_body, 0)
            pltpu.sync_copy(feat_v, feat_hbm.at[pl.ds(elem0, CB)])
            return carry

        lax.fori_loop(0, NCHUNK, chunk_body, 0)

    return k(words_flat, vectors)


def _tc_linear(feat, Wt, b2):
    BB = 2048

    def body(f_ref, w_ref, b_ref, o_ref):
        o_ref[...] = (
            jnp.dot(f_ref[...], w_ref[...], preferred_element_type=jnp.float32)
            + b_ref[...]
        )

    return pl.pallas_call(
        body,
        grid=(B // BB,),
        in_specs=[
            pl.BlockSpec((BB, D), lambda i: (i, 0)),
            pl.BlockSpec((D, C), lambda i: (0, 0)),
            pl.BlockSpec((1, C), lambda i: (0, 0)),
        ],
        out_specs=pl.BlockSpec((BB, C), lambda i: (i, 0)),
        out_shape=jax.ShapeDtypeStruct((B, C), jnp.float32),
    )(feat, Wt, b2)


def kernel(words, vectors, W, b):
    words_flat = words.reshape(-1)
    feat = _sc_gather_mean(words_flat, vectors)
    return _tc_linear(feat, W.T, b.reshape(1, C))
